# baseline (speedup 1.0000x reference)
import math

import jax
import jax.numpy as jnp
from jax import lax
from jax.experimental import pallas as pl
from jax.experimental.pallas import tpu as pltpu
from jax.experimental.pallas import tpu_sc as plsc

NUM_EMBEDDINGS = 1000000
D_MODEL = 64
SCALE = math.sqrt(D_MODEL)

B_ROWS = 4096
B_COLS = 50
N_TOTAL = B_ROWS * B_COLS

_INFO = plsc.get_sparse_core_info()
NC = _INFO.num_cores
NS = _INFO.num_subcores
NW = NC * NS
LANES = _INFO.num_lanes

VB = 128
N_FULL_BLOCKS = NUM_EMBEDDINGS // VB
TAIL_V0 = N_FULL_BLOCKS * VB
BASE_BLOCKS = N_FULL_BLOCKS // NW
EXTRA = N_FULL_BLOCKS - BASE_BLOCKS * NW

B_PER_W = B_ROWS // NW


def _pack_body(table_t, tail_p, packed, blk_v, stage_v):
    wid = lax.axis_index("s") * NC + lax.axis_index("c")
    cnt = jnp.where(wid < EXTRA, BASE_BLOCKS + 1, BASE_BLOCKS)
    start = wid * BASE_BLOCKS + jnp.minimum(wid, EXTRA)

    iota = lax.iota(jnp.int32, LANES)
    rowv = [iota + m * LANES for m in range(4)]

    def do_block(b, _):
        pltpu.sync_copy(table_t.at[:, pl.ds(b * VB, VB)], blk_v)
        for p in range(VB // 2):
            for j in range(2 * D_MODEL // LANES):
                colv = jnp.full((LANES,), 2 * p + j // 4, jnp.int32)
                vals = plsc.load_gather(blk_v, [rowv[j % 4], colv])
                stage_v[p, pl.ds(j * LANES, LANES)] = vals * SCALE
        pltpu.sync_copy(stage_v, packed.at[pl.ds(b * (VB // 2), VB // 2)])
        return 0

    lax.fori_loop(start, start + cnt, do_block, 0)

    @pl.when(wid == NW - 1)
    def _tail():
        pltpu.sync_copy(tail_p, stage_v.at[pl.ds(0, VB // 4)])
        pltpu.sync_copy(stage_v.at[pl.ds(0, VB // 4)],
                        packed.at[pl.ds(TAIL_V0 // 2, VB // 4)])


def _lookup_body(packed, idx_t, out_t, idx_v, gbuf, hbuf, rows_v, stage_v, sem):
    wid = lax.axis_index("s") * NC + lax.axis_index("c")
    b0 = wid * B_PER_W

    iota = lax.iota(jnp.int32, LANES)

    def do_t(t, _):
        pltpu.sync_copy(idx_t.at[pl.ds(t * B_ROWS + b0, B_PER_W)], idx_v)
        for k in range(B_PER_W // LANES):
            v = idx_v[pl.ds(k * LANES, LANES)]
            gbuf[pl.ds(k * LANES, LANES)] = lax.shift_right_logical(v, 1)
            hbuf[pl.ds(k * LANES, LANES)] = (v & 1) * D_MODEL
        pltpu.async_copy(packed.at[gbuf], rows_v, sem).wait()
        for k in range(B_PER_W // LANES):
            hv = hbuf[pl.ds(k * LANES, LANES)]
            rv = iota + k * LANES

            def do_d(d, _):
                vals = plsc.load_gather(rows_v, [rv, hv + d])
                stage_v[d, pl.ds(k * LANES, LANES)] = vals
                return 0

            lax.fori_loop(0, D_MODEL, do_d, 0)
        pltpu.sync_copy(stage_v, out_t.at[t, :, pl.ds(b0, B_PER_W)])
        return 0

    lax.fori_loop(0, B_COLS, do_t, 0)


@jax.jit
def _embed(table_t, tail_p, idx_t):
    mesh = plsc.VectorSubcoreMesh(core_axis_name="c", subcore_axis_name="s")
    params = pltpu.CompilerParams(use_tc_tiling_on_sc=True,
                                  needs_layout_passes=False)
    packed = pl.kernel(
        _pack_body,
        out_type=jax.ShapeDtypeStruct((NUM_EMBEDDINGS // 2, 2 * D_MODEL),
                                      jnp.float32),
        mesh=mesh,
        scratch_types=[
            pltpu.VMEM((D_MODEL, VB), jnp.float32),
            pltpu.VMEM((VB // 2, 2 * D_MODEL), jnp.float32),
        ],
        compiler_params=params,
    )(table_t, tail_p)
    out_t = pl.kernel(
        _lookup_body,
        out_type=jax.ShapeDtypeStruct((B_COLS, D_MODEL, B_ROWS), jnp.float32),
        mesh=mesh,
        scratch_types=[
            pltpu.VMEM((B_PER_W,), jnp.int32),
            pltpu.VMEM((B_PER_W,), jnp.int32),
            pltpu.VMEM((B_PER_W,), jnp.int32),
            pltpu.VMEM((B_PER_W, 2 * D_MODEL), jnp.float32),
            pltpu.VMEM((D_MODEL, B_PER_W), jnp.float32),
            pltpu.SemaphoreType.DMA,
        ],
        compiler_params=params,
    )(packed, idx_t)
    return out_t


def kernel(inputs, table):
    table_t = table.T
    tail_p = (table[TAIL_V0:] * SCALE).reshape(VB // 4, 2 * D_MODEL)
    idx_t = inputs.T.reshape(-1).astype(jnp.int32)
    out_t = _embed(table_t, tail_p, idx_t)
    return jnp.transpose(out_t, (2, 0, 1))


# speedup vs baseline: 1.6876x; 1.6876x over previous
import math

import jax
import jax.numpy as jnp
from jax import lax
from jax.experimental import pallas as pl
from jax.experimental.pallas import tpu as pltpu
from jax.experimental.pallas import tpu_sc as plsc

NUM_EMBEDDINGS = 1000000
D_MODEL = 64
SCALE = math.sqrt(D_MODEL)

B_ROWS = 4096
B_COLS = 50
N_TOTAL = B_ROWS * B_COLS

_INFO = plsc.get_sparse_core_info()
NC = _INFO.num_cores
NS = _INFO.num_subcores
NW = NC * NS
LANES = _INFO.num_lanes

VB = 128
N_FULL_BLOCKS = NUM_EMBEDDINGS // VB
TAIL_V0 = N_FULL_BLOCKS * VB
BASE_BLOCKS = N_FULL_BLOCKS // NW
EXTRA = N_FULL_BLOCKS - BASE_BLOCKS * NW

B_PER_W = B_ROWS // NW


def _pack_body(table_t, tail_p, packed, blk_v, stage_v):
    wid = lax.axis_index("s") * NC + lax.axis_index("c")
    cnt = jnp.where(wid < EXTRA, BASE_BLOCKS + 1, BASE_BLOCKS)
    start = wid * BASE_BLOCKS + jnp.minimum(wid, EXTRA)

    iota = lax.iota(jnp.int32, LANES)
    rowv = [iota + m * LANES for m in range(4)]

    def do_block(b, _):
        pltpu.sync_copy(table_t.at[:, pl.ds(b * VB, VB)], blk_v)

        @plsc.parallel_loop(0, VB // 2, unroll=4)
        def _p(p):
            base = jnp.full((LANES,), 2 * p, jnp.int32)
            for j in range(2 * D_MODEL // LANES):
                colv = base + (j // 4)
                vals = plsc.load_gather(blk_v, [rowv[j % 4], colv])
                stage_v[p, pl.ds(j * LANES, LANES)] = vals * SCALE

        pltpu.sync_copy(stage_v, packed.at[pl.ds(b * (VB // 2), VB // 2)])
        return 0

    lax.fori_loop(start, start + cnt, do_block, 0)

    @pl.when(wid == NW - 1)
    def _tail():
        pltpu.sync_copy(tail_p, stage_v.at[pl.ds(0, VB // 4)])
        pltpu.sync_copy(stage_v.at[pl.ds(0, VB // 4)],
                        packed.at[pl.ds(TAIL_V0 // 2, VB // 4)])


def _lookup_body(packed, idx_t, out_t, idx_v, gbuf, hbuf, rows_v, stage_v, sem):
    wid = lax.axis_index("s") * NC + lax.axis_index("c")
    b0 = wid * B_PER_W

    iota = lax.iota(jnp.int32, LANES)

    def do_t(t, _):
        pltpu.sync_copy(idx_t.at[pl.ds(t * B_ROWS + b0, B_PER_W)], idx_v)
        for k in range(B_PER_W // LANES):
            v = idx_v[pl.ds(k * LANES, LANES)]
            gbuf[pl.ds(k * LANES, LANES)] = lax.shift_right_logical(v, 1)
            hbuf[pl.ds(k * LANES, LANES)] = (v & 1) * D_MODEL
        pltpu.async_copy(packed.at[gbuf], rows_v, sem).wait()
        for k in range(B_PER_W // LANES):
            hv = hbuf[pl.ds(k * LANES, LANES)]
            rv = iota + k * LANES

            @plsc.parallel_loop(0, D_MODEL, unroll=8)
            def _d(d, _hv=hv, _rv=rv, _k=k):
                vals = plsc.load_gather(rows_v, [_rv, _hv + d])
                stage_v[d, pl.ds(_k * LANES, LANES)] = vals
        pltpu.sync_copy(stage_v, out_t.at[t, :, pl.ds(b0, B_PER_W)])
        return 0

    lax.fori_loop(0, B_COLS, do_t, 0)


@jax.jit
def _embed(table_t, tail_p, idx_t):
    mesh = plsc.VectorSubcoreMesh(core_axis_name="c", subcore_axis_name="s")
    params = pltpu.CompilerParams(use_tc_tiling_on_sc=True,
                                  needs_layout_passes=False)
    packed = pl.kernel(
        _pack_body,
        out_type=jax.ShapeDtypeStruct((NUM_EMBEDDINGS // 2, 2 * D_MODEL),
                                      jnp.float32),
        mesh=mesh,
        scratch_types=[
            pltpu.VMEM((D_MODEL, VB), jnp.float32),
            pltpu.VMEM((VB // 2, 2 * D_MODEL), jnp.float32),
        ],
        compiler_params=params,
    )(table_t, tail_p)
    out_t = pl.kernel(
        _lookup_body,
        out_type=jax.ShapeDtypeStruct((B_COLS, D_MODEL, B_ROWS), jnp.float32),
        mesh=mesh,
        scratch_types=[
            pltpu.VMEM((B_PER_W,), jnp.int32),
            pltpu.VMEM((B_PER_W,), jnp.int32),
            pltpu.VMEM((B_PER_W,), jnp.int32),
            pltpu.VMEM((B_PER_W, 2 * D_MODEL), jnp.float32),
            pltpu.VMEM((D_MODEL, B_PER_W), jnp.float32),
            pltpu.SemaphoreType.DMA,
        ],
        compiler_params=params,
    )(packed, idx_t)
    return out_t


def kernel(inputs, table):
    table_t = table.T
    tail_p = (table[TAIL_V0:] * SCALE).reshape(VB // 4, 2 * D_MODEL)
    idx_t = inputs.T.reshape(-1).astype(jnp.int32)
    out_t = _embed(table_t, tail_p, idx_t)
    return jnp.transpose(out_t, (2, 0, 1))


# speedup vs baseline: 3.1178x; 1.8475x over previous
import math

import jax
import jax.numpy as jnp
from jax import lax
from jax.experimental import pallas as pl
from jax.experimental.pallas import tpu as pltpu
from jax.experimental.pallas import tpu_sc as plsc

NUM_EMBEDDINGS = 1000000
D_MODEL = 64
SCALE = math.sqrt(D_MODEL)

B_ROWS = 4096
B_COLS = 50
N_TOTAL = B_ROWS * B_COLS

_INFO = plsc.get_sparse_core_info()
NC = _INFO.num_cores
NS = _INFO.num_subcores
NW = NC * NS
LANES = _INFO.num_lanes

SUB = 128
K_PER_CHUNK = 5
CHUNK = SUB * K_PER_CHUNK
PER_W = N_TOTAL // NW
NCHUNK = PER_W // CHUNK
ROWS_PER_W = PER_W // SUB


def _body(table_hbm, idx_hbm, out_hbm, idx_v, buf0, buf1,
          sem_g0, sem_g1, sem_w0, sem_w1):
    wid = lax.axis_index("s") * NC + lax.axis_index("c")
    out_base = wid * PER_W

    pltpu.sync_copy(idx_hbm.at[wid], idx_v)

    bufs = (buf0, buf1)
    sems_g = (sem_g0, sem_g1)
    sems_w = (sem_w0, sem_w1)

    def fire(ci, b):
        for j in range(K_PER_CHUNK):
            pltpu.async_copy(table_hbm.at[idx_v.at[ci * K_PER_CHUNK + j]],
                             bufs[b].at[pl.ds(j * SUB, SUB)], sems_g[b])

    def drain(ci, b):
        for j in range(K_PER_CHUNK):
            pltpu.make_async_copy(table_hbm.at[idx_v.at[ci * K_PER_CHUNK + j]],
                                  bufs[b].at[pl.ds(j * SUB, SUB)],
                                  sems_g[b]).wait()

    def scale(b):
        buf = bufs[b]

        @plsc.parallel_loop(0, CHUNK, unroll=8)
        def _r(r):
            for v in range(D_MODEL // LANES):
                sl = pl.ds(v * LANES, LANES)
                buf[r, sl] = buf[r, sl] * SCALE

    def out_start(ci, b):
        pltpu.async_copy(bufs[b],
                         out_hbm.at[pl.ds(out_base + ci * CHUNK, CHUNK)],
                         sems_w[b])

    def out_wait(ci, b):
        pltpu.make_async_copy(bufs[b],
                              out_hbm.at[pl.ds(out_base + ci * CHUNK, CHUNK)],
                              sems_w[b]).wait()

    fire(0, 0)
    fire(1, 1)

    def pair(p, _):
        for b in range(2):
            ci = 2 * p + b
            drain(ci, b)
            scale(b)
            out_start(ci, b)
        for b in range(2):
            ci = 2 * p + b
            out_wait(ci, b)
            fire(ci + 2, b)
        return 0

    lax.fori_loop(0, NCHUNK // 2 - 1, pair, 0)

    for b in range(2):
        ci = NCHUNK - 2 + b
        drain(ci, b)
        scale(b)
        out_start(ci, b)
    for b in range(2):
        out_wait(NCHUNK - 2 + b, b)


@jax.jit
def _embed(table, idx3d):
    mesh = plsc.VectorSubcoreMesh(core_axis_name="c", subcore_axis_name="s")
    kern = pl.kernel(
        _body,
        out_type=jax.ShapeDtypeStruct((N_TOTAL, D_MODEL), jnp.float32),
        mesh=mesh,
        scratch_types=[
            pltpu.VMEM((ROWS_PER_W, SUB), jnp.int32),
            pltpu.VMEM((CHUNK, D_MODEL), jnp.float32),
            pltpu.VMEM((CHUNK, D_MODEL), jnp.float32),
            pltpu.SemaphoreType.DMA,
            pltpu.SemaphoreType.DMA,
            pltpu.SemaphoreType.DMA,
            pltpu.SemaphoreType.DMA,
        ],
        compiler_params=pltpu.CompilerParams(use_tc_tiling_on_sc=False),
    )
    return kern(table, idx3d)


def kernel(inputs, table):
    idx3d = inputs.reshape(NW, ROWS_PER_W, SUB).astype(jnp.int32)
    out = _embed(table, idx3d)
    return out.reshape(B_ROWS, B_COLS, D_MODEL)
